# baseline (speedup 1.0000x reference)
import functools

import jax
import jax.numpy as jnp
from jax import lax
from jax.experimental import pallas as pl
from jax.experimental.pallas import tpu as pltpu
from jax.experimental.pallas import tpu_sc as plsc

_K = 32
_CUTOFF = 10.0
_N = 4096
_L = 16
_NW = 32
_RPW = _N // _NW
_BITS100 = 0x42C80000
_HUGE = 0x7F000000


def _bsearch16(bt_ref, keys):
    lo = jnp.zeros((_L,), jnp.int32)
    hi = jnp.full((_L,), _N, jnp.int32)
    for _ in range(12):
        mid = (lo + hi) >> 1
        vals = plsc.load_gather(bt_ref, [mid])
        cond = vals < keys
        lo = jnp.where(cond, mid + 1, lo)
        hi = jnp.where(cond, hi, mid)
    return lo


def _merge32(c1k, c1v, c2k, c2v):
    c = c2k < c1k
    lk = jnp.where(c, c2k, c1k)
    lv = jnp.where(c, c2v, c1v)
    hk = jnp.where(c, c1k, c2k)
    hv = jnp.where(c, c1v, c2v)
    lk, lv = plsc.sort_key_val(lk, lv, descending=True)
    hk, hv = plsc.sort_key_val(hk, hv, descending=True)
    return lk, lv, hk, hv


def _tmerge(t, c0k, c0v, c1k, c1v):
    t0k, t0v, t1k, t1v = t
    c = c1k < t0k
    l0k = jnp.where(c, c1k, t0k)
    l0v = jnp.where(c, c1v, t0v)
    c = c0k < t1k
    l1k = jnp.where(c, c0k, t1k)
    l1v = jnp.where(c, c0v, t1v)
    c = l1k < l0k
    b0k = jnp.where(c, l1k, l0k)
    b0v = jnp.where(c, l1v, l0v)
    b1k = jnp.where(c, l0k, l1k)
    b1v = jnp.where(c, l0v, l1v)
    b0k, b0v = plsc.sort_key_val(b0k, b0v)
    b1k, b1v = plsc.sort_key_val(b1k, b1v)
    return b0k, b0v, b1k, b1v


def _sqrt16(s):
    sc = jnp.maximum(s, 1e-30)
    b = plsc.bitcast(sc, jnp.int32)
    y = plsc.bitcast(0x5F3759DF - (b >> 1), jnp.float32)
    h = 0.5 * sc
    y = y * (1.5 - h * y * y)
    y = y * (1.5 - h * y * y)
    w = sc * y
    w = 0.5 * (w + sc / w)
    return jnp.where(s > 0.0, w, 0.0)


def _sc_topk():
    mesh = plsc.VectorSubcoreMesh(core_axis_name="c", subcore_axis_name="s")

    @functools.partial(
        pl.kernel,
        mesh=mesh,
        compiler_params=pltpu.CompilerParams(needs_layout_passes=False),
        out_type=[
            jax.ShapeDtypeStruct((2, _N * _K), jnp.int32),
            jax.ShapeDtypeStruct((_N * _K,), jnp.float32),
        ],
        scratch_types=[
            pltpu.VMEM((_N + _L,), jnp.float32),
            pltpu.VMEM((_N + _L,), jnp.float32),
            pltpu.VMEM((_N + _L,), jnp.float32),
            pltpu.VMEM((_N + _L,), jnp.int32),
            pltpu.VMEM((_RPW + _L,), jnp.int32),
            pltpu.VMEM((_RPW + _L,), jnp.int32),
            pltpu.VMEM((_RPW * _K,), jnp.float32),
            pltpu.VMEM((_RPW * _K,), jnp.int32),
            pltpu.VMEM((_RPW * _K,), jnp.int32),
        ],
    )
    def kfn(xs_h, ys_h, zs_h, bt_h, ei_h, ew_h,
            xs, ys, zs, bt, seg_s, seg_e, ow, oidx, orow):
        wid = lax.axis_index("c") * 16 + lax.axis_index("s")
        base = wid * _RPW

        pltpu.sync_copy(xs_h, xs.at[pl.ds(0, _N)])
        pltpu.sync_copy(ys_h, ys.at[pl.ds(0, _N)])
        pltpu.sync_copy(zs_h, zs.at[pl.ds(0, _N)])
        pltpu.sync_copy(bt_h, bt.at[pl.ds(0, _N)])

        lane = lax.iota(jnp.int32, _L)
        for g in range(_RPW // _L):
            bb = bt[pl.ds(base + g * _L, _L)]
            seg_s[pl.ds(g * _L, _L)] = _bsearch16(bt, bb)
            seg_e[pl.ds(g * _L, _L)] = _bsearch16(bt, bb + 1)

        _G = 4

        def group_body(p, _):
            r0 = _G * p
            i0 = base + r0
            sv = seg_s[pl.ds(r0, _L)]
            ev = seg_e[pl.ds(r0, _L)]
            vx = xs[pl.ds(i0, _L)]
            vy = ys[pl.ds(i0, _L)]
            vz = zs[pl.ds(i0, _L)]
            rows = [
                (i0 + q, sv[q], ev[q], vx[q], vy[q], vz[q])
                for q in range(_G)
            ]

            def keys_for(row, cx, cy, cz, jvec):
                i, s_r, e_r, bx, by, bz = row
                dx = bx - cx
                dy = by - cy
                dz = bz - cz
                sq = dx * dx + dy * dy + dz * dz
                kbits = plsc.bitcast(sq, jnp.int32)
                masked = (
                    (jvec < s_r) | (jvec >= e_r) | (jvec == i)
                    | (sq >= _CUTOFF * _CUTOFF)
                )
                return jnp.where(masked, _BITS100 + jvec, kbits), jvec

            def super_chunk(j0, ts):
                j1 = j0 + _L
                cx1 = xs[pl.ds(j0, _L)]
                cy1 = ys[pl.ds(j0, _L)]
                cz1 = zs[pl.ds(j0, _L)]
                cx2 = xs[pl.ds(j1, _L)]
                cy2 = ys[pl.ds(j1, _L)]
                cz2 = zs[pl.ds(j1, _L)]
                jv1 = lane + j0
                jv2 = lane + j1
                outs = []
                for row, t in zip(rows, ts):
                    k1, v1 = keys_for(row, cx1, cy1, cz1, jv1)
                    k2, v2 = keys_for(row, cx2, cy2, cz2, jv2)
                    k1, v1 = plsc.sort_key_val(k1, v1)
                    k2, v2 = plsc.sort_key_val(k2, v2, descending=True)
                    c0k, c0v, c1k, c1v = _merge32(k1, v1, k2, v2)
                    outs.append(tuple(_tmerge(t, c0k, c0v, c1k, c1v)))
                return tuple(outs)

            init = (
                jnp.full((_L,), _HUGE, jnp.int32),
                jnp.zeros((_L,), jnp.int32),
                jnp.full((_L,), _HUGE, jnp.int32),
                jnp.zeros((_L,), jnp.int32),
            )

            s_grp = rows[0][1]
            e_grp = rows[0][2]
            for q in range(1, _G):
                s_grp = jnp.minimum(s_grp, rows[q][1])
                e_grp = jnp.maximum(e_grp, rows[q][2])
            c_lo = jnp.maximum(s_grp >> 5, 2)
            c_hi = jnp.maximum((e_grp + 31) >> 5, 2)

            def seg_body(c, ts):
                return super_chunk(pl.multiple_of(c * 2 * _L, 2 * _L), ts)

            ts = lax.fori_loop(c_lo, c_hi, seg_body, (init,) * _G)

            def do_prefix(ts):
                return super_chunk(2 * _L, super_chunk(0, ts))

            need = s_grp < 4 * _L
            for q in range(_G):
                need = need | (ts[q][2][_L - 1] >= _BITS100)
            ts = lax.cond(need, do_prefix, lambda ts: ts, ts)

            for q in range(_G):
                t0k, t0v, t1k, t1v = ts[q]
                w0 = jnp.where(
                    t0k >= _BITS100, _CUTOFF,
                    _sqrt16(plsc.bitcast(t0k, jnp.float32)))
                w1 = jnp.where(
                    t1k >= _BITS100, _CUTOFF,
                    _sqrt16(plsc.bitcast(t1k, jnp.float32)))
                o = pl.multiple_of((r0 + q) * _K, _K)
                ow[pl.ds(o, _L)] = w0
                ow[pl.ds(o + _L, _L)] = w1
                oidx[pl.ds(o, _L)] = t0v
                oidx[pl.ds(o + _L, _L)] = t1v
                ivec = jnp.full((_L,), i0 + q, jnp.int32)
                orow[pl.ds(o, _L)] = ivec
                orow[pl.ds(o + _L, _L)] = ivec
            return 0

        lax.fori_loop(0, _RPW // _G, group_body, 0)

        pltpu.sync_copy(oidx, ei_h.at[0, pl.ds(base * _K, _RPW * _K)])
        pltpu.sync_copy(orow, ei_h.at[1, pl.ds(base * _K, _RPW * _K)])
        pltpu.sync_copy(ow, ew_h.at[pl.ds(base * _K, _RPW * _K)])

    return kfn


@jax.jit
def kernel(pos, batch):
    n = pos.shape[0]
    pos = pos.astype(jnp.float32)
    batch = batch.astype(jnp.int32)

    edge_index, edge_weight = _sc_topk()(pos[:, 0], pos[:, 1], pos[:, 2], batch)
    return edge_index, edge_weight


# speedup vs baseline: 1.0832x; 1.0832x over previous
import functools

import jax
import jax.numpy as jnp
from jax import lax
from jax.experimental import pallas as pl
from jax.experimental.pallas import tpu as pltpu
from jax.experimental.pallas import tpu_sc as plsc

_K = 32
_CUTOFF = 10.0
_N = 4096
_L = 16
_NW = 32
_RPW = _N // _NW
_BITS100 = 0x42C80000
_HUGE = 0x7F000000


def _bsearch16(bt_ref, keys):
    lo = jnp.zeros((_L,), jnp.int32)
    hi = jnp.full((_L,), _N, jnp.int32)
    for _ in range(12):
        mid = (lo + hi) >> 1
        vals = plsc.load_gather(bt_ref, [mid])
        cond = vals < keys
        lo = jnp.where(cond, mid + 1, lo)
        hi = jnp.where(cond, hi, mid)
    return lo


def _merge32(c1k, c1v, c2k, c2v):
    c = c2k < c1k
    lk = jnp.where(c, c2k, c1k)
    lv = jnp.where(c, c2v, c1v)
    hk = jnp.where(c, c1k, c2k)
    hv = jnp.where(c, c1v, c2v)
    lk, lv = plsc.sort_key_val(lk, lv, descending=True)
    hk, hv = plsc.sort_key_val(hk, hv, descending=True)
    return lk, lv, hk, hv


def _tmerge(t, c0k, c0v, c1k, c1v):
    t0k, t0v, t1k, t1v = t
    c = c1k < t0k
    l0k = jnp.where(c, c1k, t0k)
    l0v = jnp.where(c, c1v, t0v)
    c = c0k < t1k
    l1k = jnp.where(c, c0k, t1k)
    l1v = jnp.where(c, c0v, t1v)
    c = l1k < l0k
    b0k = jnp.where(c, l1k, l0k)
    b0v = jnp.where(c, l1v, l0v)
    b1k = jnp.where(c, l0k, l1k)
    b1v = jnp.where(c, l0v, l1v)
    b0k, b0v = plsc.sort_key_val(b0k, b0v)
    b1k, b1v = plsc.sort_key_val(b1k, b1v)
    return b0k, b0v, b1k, b1v


def _sqrt16(s):
    sc = jnp.maximum(s, 1e-30)
    b = plsc.bitcast(sc, jnp.int32)
    y = plsc.bitcast(0x5F3759DF - (b >> 1), jnp.float32)
    h = 0.5 * sc
    y = y * (1.5 - h * y * y)
    y = y * (1.5 - h * y * y)
    w = sc * y
    w = 0.5 * (w + sc / w)
    return jnp.where(s > 0.0, w, 0.0)


def _sc_topk():
    mesh = plsc.VectorSubcoreMesh(core_axis_name="c", subcore_axis_name="s")

    @functools.partial(
        pl.kernel,
        mesh=mesh,
        compiler_params=pltpu.CompilerParams(needs_layout_passes=False),
        out_type=[
            jax.ShapeDtypeStruct((2, _N * _K), jnp.int32),
            jax.ShapeDtypeStruct((_N * _K,), jnp.float32),
        ],
        scratch_types=[
            pltpu.VMEM((_N + _L,), jnp.float32),
            pltpu.VMEM((_N + _L,), jnp.float32),
            pltpu.VMEM((_N + _L,), jnp.float32),
            pltpu.VMEM((_N + _L,), jnp.int32),
            pltpu.VMEM((_RPW + _L,), jnp.int32),
            pltpu.VMEM((_RPW + _L,), jnp.int32),
            pltpu.VMEM((_RPW * _K,), jnp.float32),
            pltpu.VMEM((_RPW * _K,), jnp.int32),
            pltpu.VMEM((_RPW * _K,), jnp.int32),
        ],
    )
    def kfn(xs_h, ys_h, zs_h, bt_h, ei_h, ew_h,
            xs, ys, zs, bt, seg_s, seg_e, ow, oidx, orow):
        wid = lax.axis_index("c") * 16 + lax.axis_index("s")
        base = wid * _RPW

        pltpu.sync_copy(xs_h, xs.at[pl.ds(0, _N)])
        pltpu.sync_copy(ys_h, ys.at[pl.ds(0, _N)])
        pltpu.sync_copy(zs_h, zs.at[pl.ds(0, _N)])
        pltpu.sync_copy(bt_h, bt.at[pl.ds(0, _N)])

        lane = lax.iota(jnp.int32, _L)
        for g in range(_RPW // _L):
            bb = bt[pl.ds(base + g * _L, _L)]
            seg_s[pl.ds(g * _L, _L)] = _bsearch16(bt, bb)
            seg_e[pl.ds(g * _L, _L)] = _bsearch16(bt, bb + 1)

        _G = 2

        def group_body(p, _):
            r0 = _G * p
            i0 = base + r0
            sv = seg_s[pl.ds(r0, _L)]
            ev = seg_e[pl.ds(r0, _L)]
            vx = xs[pl.ds(i0, _L)]
            vy = ys[pl.ds(i0, _L)]
            vz = zs[pl.ds(i0, _L)]
            rows = [
                (i0 + q, sv[q], ev[q], vx[q], vy[q], vz[q])
                for q in range(_G)
            ]

            def keys_for(row, cx, cy, cz, jvec):
                i, s_r, e_r, bx, by, bz = row
                dx = bx - cx
                dy = by - cy
                dz = bz - cz
                sq = dx * dx + dy * dy + dz * dz
                kbits = plsc.bitcast(sq, jnp.int32)
                masked = (
                    (jvec < s_r) | (jvec >= e_r) | (jvec == i)
                    | (sq >= _CUTOFF * _CUTOFF)
                )
                return jnp.where(masked, _BITS100 + jvec, kbits), jvec

            def super_chunk(j0, ts):
                j1 = j0 + _L
                cx1 = xs[pl.ds(j0, _L)]
                cy1 = ys[pl.ds(j0, _L)]
                cz1 = zs[pl.ds(j0, _L)]
                cx2 = xs[pl.ds(j1, _L)]
                cy2 = ys[pl.ds(j1, _L)]
                cz2 = zs[pl.ds(j1, _L)]
                jv1 = lane + j0
                jv2 = lane + j1
                outs = []
                for row, t in zip(rows, ts):
                    k1, v1 = keys_for(row, cx1, cy1, cz1, jv1)
                    k2, v2 = keys_for(row, cx2, cy2, cz2, jv2)
                    k1, v1 = plsc.sort_key_val(k1, v1)
                    k2, v2 = plsc.sort_key_val(k2, v2, descending=True)
                    c0k, c0v, c1k, c1v = _merge32(k1, v1, k2, v2)
                    outs.append(tuple(_tmerge(t, c0k, c0v, c1k, c1v)))
                return tuple(outs)

            init = (
                jnp.full((_L,), _HUGE, jnp.int32),
                jnp.zeros((_L,), jnp.int32),
                jnp.full((_L,), _HUGE, jnp.int32),
                jnp.zeros((_L,), jnp.int32),
            )

            s_grp = rows[0][1]
            e_grp = rows[0][2]
            for q in range(1, _G):
                s_grp = jnp.minimum(s_grp, rows[q][1])
                e_grp = jnp.maximum(e_grp, rows[q][2])
            c_lo = jnp.maximum(s_grp >> 5, 2)
            c_hi = jnp.maximum((e_grp + 31) >> 5, 2)

            def seg_body(c, ts):
                return super_chunk(pl.multiple_of(c * 2 * _L, 2 * _L), ts)

            ts = lax.fori_loop(c_lo, c_hi, seg_body, (init,) * _G)

            def do_prefix(ts):
                return super_chunk(2 * _L, super_chunk(0, ts))

            need = s_grp < 4 * _L
            for q in range(_G):
                need = need | (ts[q][2][_L - 1] >= _BITS100)
            ts = lax.cond(need, do_prefix, lambda ts: ts, ts)

            for q in range(_G):
                t0k, t0v, t1k, t1v = ts[q]
                w0 = jnp.where(
                    t0k >= _BITS100, _CUTOFF,
                    _sqrt16(plsc.bitcast(t0k, jnp.float32)))
                w1 = jnp.where(
                    t1k >= _BITS100, _CUTOFF,
                    _sqrt16(plsc.bitcast(t1k, jnp.float32)))
                o = pl.multiple_of((r0 + q) * _K, _K)
                ow[pl.ds(o, _L)] = w0
                ow[pl.ds(o + _L, _L)] = w1
                oidx[pl.ds(o, _L)] = t0v
                oidx[pl.ds(o + _L, _L)] = t1v
                ivec = jnp.full((_L,), i0 + q, jnp.int32)
                orow[pl.ds(o, _L)] = ivec
                orow[pl.ds(o + _L, _L)] = ivec
            return 0

        lax.fori_loop(0, _RPW // _G, group_body, 0)

        pltpu.sync_copy(oidx, ei_h.at[0, pl.ds(base * _K, _RPW * _K)])
        pltpu.sync_copy(orow, ei_h.at[1, pl.ds(base * _K, _RPW * _K)])
        pltpu.sync_copy(ow, ew_h.at[pl.ds(base * _K, _RPW * _K)])

    return kfn


@jax.jit
def kernel(pos, batch):
    n = pos.shape[0]
    pos = pos.astype(jnp.float32)
    batch = batch.astype(jnp.int32)

    edge_index, edge_weight = _sc_topk()(pos[:, 0], pos[:, 1], pos[:, 2], batch)
    return edge_index, edge_weight


# speedup vs baseline: 1.1697x; 1.0798x over previous
import functools

import jax
import jax.numpy as jnp
from jax import lax
from jax.experimental import pallas as pl
from jax.experimental.pallas import tpu as pltpu
from jax.experimental.pallas import tpu_sc as plsc

_K = 32
_CUTOFF = 10.0
_N = 4096
_L = 16
_NW = 32
_RPW = _N // _NW
_BITS100 = 0x42C80000
_HUGE = 0x7F000000


def _bsearch16_2(bt_ref, keys_a, keys_b):
    lo_a = jnp.zeros((_L,), jnp.int32)
    hi_a = jnp.full((_L,), _N, jnp.int32)
    lo_b = jnp.zeros((_L,), jnp.int32)
    hi_b = jnp.full((_L,), _N, jnp.int32)
    for _ in range(12):
        mid_a = (lo_a + hi_a) >> 1
        mid_b = (lo_b + hi_b) >> 1
        va = plsc.load_gather(bt_ref, [mid_a])
        vb = plsc.load_gather(bt_ref, [mid_b])
        ca = va < keys_a
        cb = vb < keys_b
        lo_a = jnp.where(ca, mid_a + 1, lo_a)
        hi_a = jnp.where(ca, hi_a, mid_a)
        lo_b = jnp.where(cb, mid_b + 1, lo_b)
        hi_b = jnp.where(cb, hi_b, mid_b)
    return lo_a, lo_b


def _merge32(c1k, c1v, c2k, c2v):
    c = c2k < c1k
    lk = jnp.where(c, c2k, c1k)
    lv = jnp.where(c, c2v, c1v)
    hk = jnp.where(c, c1k, c2k)
    hv = jnp.where(c, c1v, c2v)
    lk, lv = plsc.sort_key_val(lk, lv, descending=True)
    hk, hv = plsc.sort_key_val(hk, hv, descending=True)
    return lk, lv, hk, hv


def _tmerge(t, c0k, c0v, c1k, c1v):
    t0k, t0v, t1k, t1v = t
    c = c1k < t0k
    l0k = jnp.where(c, c1k, t0k)
    l0v = jnp.where(c, c1v, t0v)
    c = c0k < t1k
    l1k = jnp.where(c, c0k, t1k)
    l1v = jnp.where(c, c0v, t1v)
    c = l1k < l0k
    b0k = jnp.where(c, l1k, l0k)
    b0v = jnp.where(c, l1v, l0v)
    b1k = jnp.where(c, l0k, l1k)
    b1v = jnp.where(c, l0v, l1v)
    b0k, b0v = plsc.sort_key_val(b0k, b0v)
    b1k, b1v = plsc.sort_key_val(b1k, b1v)
    return b0k, b0v, b1k, b1v


def _sqrt16(s):
    sc = jnp.maximum(s, 1e-30)
    b = plsc.bitcast(sc, jnp.int32)
    y = plsc.bitcast(0x5F3759DF - (b >> 1), jnp.float32)
    h = 0.5 * sc
    y = y * (1.5 - h * y * y)
    y = y * (1.5 - h * y * y)
    w = sc * y
    w = 0.5 * (w + sc / w)
    return jnp.where(s > 0.0, w, 0.0)


def _sc_topk():
    mesh = plsc.VectorSubcoreMesh(core_axis_name="c", subcore_axis_name="s")

    @functools.partial(
        pl.kernel,
        mesh=mesh,
        compiler_params=pltpu.CompilerParams(needs_layout_passes=False),
        out_type=[
            jax.ShapeDtypeStruct((2, _N * _K), jnp.int32),
            jax.ShapeDtypeStruct((_N * _K,), jnp.float32),
        ],
        scratch_types=[
            pltpu.VMEM((_N + _L,), jnp.float32),
            pltpu.VMEM((_N + _L,), jnp.float32),
            pltpu.VMEM((_N + _L,), jnp.float32),
            pltpu.VMEM((_N + _L,), jnp.int32),
            pltpu.VMEM((_RPW + _L,), jnp.int32),
            pltpu.VMEM((_RPW + _L,), jnp.int32),
            pltpu.VMEM((_RPW * _K,), jnp.float32),
            pltpu.VMEM((_RPW * _K,), jnp.int32),
            pltpu.VMEM((_RPW * _K,), jnp.int32),
            pltpu.SemaphoreType.DMA,
        ],
    )
    def kfn(xs_h, ys_h, zs_h, bt_h, ei_h, ew_h,
            xs, ys, zs, bt, seg_s, seg_e, ow, oidx, orow, dsem):
        wid = lax.axis_index("c") * 16 + lax.axis_index("s")
        base = wid * _RPW

        cx = pltpu.async_copy(xs_h, xs.at[pl.ds(0, _N)], dsem)
        cy = pltpu.async_copy(ys_h, ys.at[pl.ds(0, _N)], dsem)
        cz = pltpu.async_copy(zs_h, zs.at[pl.ds(0, _N)], dsem)
        pltpu.sync_copy(bt_h, bt.at[pl.ds(0, _N)])

        lane = lax.iota(jnp.int32, _L)
        for g in range(_RPW // _L):
            bb = bt[pl.ds(base + g * _L, _L)]
            ls, le = _bsearch16_2(bt, bb, bb + 1)
            seg_s[pl.ds(g * _L, _L)] = ls
            seg_e[pl.ds(g * _L, _L)] = le
        cx.wait()
        cy.wait()
        cz.wait()

        _G = 2

        def group_body(p, _):
            r0 = _G * p
            i0 = base + r0
            sv = seg_s[pl.ds(r0, _L)]
            ev = seg_e[pl.ds(r0, _L)]
            vx = xs[pl.ds(i0, _L)]
            vy = ys[pl.ds(i0, _L)]
            vz = zs[pl.ds(i0, _L)]
            rows = [
                (i0 + q, sv[q], ev[q], vx[q], vy[q], vz[q])
                for q in range(_G)
            ]

            def keys_for(row, cx, cy, cz, jvec):
                i, s_r, e_r, bx, by, bz = row
                dx = bx - cx
                dy = by - cy
                dz = bz - cz
                sq = dx * dx + dy * dy + dz * dz
                kbits = plsc.bitcast(sq, jnp.int32)
                masked = (
                    (jvec < s_r) | (jvec >= e_r) | (jvec == i)
                    | (sq >= _CUTOFF * _CUTOFF)
                )
                return jnp.where(masked, _BITS100 + jvec, kbits), jvec

            def super_chunk(j0, ts):
                j1 = j0 + _L
                cx1 = xs[pl.ds(j0, _L)]
                cy1 = ys[pl.ds(j0, _L)]
                cz1 = zs[pl.ds(j0, _L)]
                cx2 = xs[pl.ds(j1, _L)]
                cy2 = ys[pl.ds(j1, _L)]
                cz2 = zs[pl.ds(j1, _L)]
                jv1 = lane + j0
                jv2 = lane + j1
                outs = []
                for row, t in zip(rows, ts):
                    k1, v1 = keys_for(row, cx1, cy1, cz1, jv1)
                    k2, v2 = keys_for(row, cx2, cy2, cz2, jv2)
                    k1, v1 = plsc.sort_key_val(k1, v1)
                    k2, v2 = plsc.sort_key_val(k2, v2, descending=True)
                    c0k, c0v, c1k, c1v = _merge32(k1, v1, k2, v2)
                    outs.append(tuple(_tmerge(t, c0k, c0v, c1k, c1v)))
                return tuple(outs)

            init = (
                jnp.full((_L,), _HUGE, jnp.int32),
                jnp.zeros((_L,), jnp.int32),
                jnp.full((_L,), _HUGE, jnp.int32),
                jnp.zeros((_L,), jnp.int32),
            )

            s_grp = rows[0][1]
            e_grp = rows[0][2]
            for q in range(1, _G):
                s_grp = jnp.minimum(s_grp, rows[q][1])
                e_grp = jnp.maximum(e_grp, rows[q][2])
            c_lo = jnp.maximum(s_grp >> 5, 2)
            c_hi = jnp.maximum((e_grp + 31) >> 5, 2)

            def seg_body(c, ts):
                return super_chunk(pl.multiple_of(c * 2 * _L, 2 * _L), ts)

            ts = lax.fori_loop(c_lo, c_hi, seg_body, (init,) * _G)

            def do_prefix(ts):
                return super_chunk(2 * _L, super_chunk(0, ts))

            need = s_grp < 4 * _L
            for q in range(_G):
                need = need | (ts[q][2][_L - 1] >= _BITS100)
            ts = lax.cond(need, do_prefix, lambda ts: ts, ts)

            for q in range(_G):
                t0k, t0v, t1k, t1v = ts[q]
                w0 = jnp.where(
                    t0k >= _BITS100, _CUTOFF,
                    _sqrt16(plsc.bitcast(t0k, jnp.float32)))
                w1 = jnp.where(
                    t1k >= _BITS100, _CUTOFF,
                    _sqrt16(plsc.bitcast(t1k, jnp.float32)))
                o = pl.multiple_of((r0 + q) * _K, _K)
                ow[pl.ds(o, _L)] = w0
                ow[pl.ds(o + _L, _L)] = w1
                oidx[pl.ds(o, _L)] = t0v
                oidx[pl.ds(o + _L, _L)] = t1v
                ivec = jnp.full((_L,), i0 + q, jnp.int32)
                orow[pl.ds(o, _L)] = ivec
                orow[pl.ds(o + _L, _L)] = ivec
            return 0

        lax.fori_loop(0, _RPW // _G, group_body, 0)

        pltpu.sync_copy(oidx, ei_h.at[0, pl.ds(base * _K, _RPW * _K)])
        pltpu.sync_copy(orow, ei_h.at[1, pl.ds(base * _K, _RPW * _K)])
        pltpu.sync_copy(ow, ew_h.at[pl.ds(base * _K, _RPW * _K)])

    return kfn


@jax.jit
def kernel(pos, batch):
    n = pos.shape[0]
    pos = pos.astype(jnp.float32)
    batch = batch.astype(jnp.int32)

    edge_index, edge_weight = _sc_topk()(pos[:, 0], pos[:, 1], pos[:, 2], batch)
    return edge_index, edge_weight
